# baseline (speedup 1.0000x reference)
import functools

import jax
import jax.numpy as jnp
from jax import lax
from jax.experimental import pallas as pl
from jax.experimental.pallas import tpu as pltpu
from jax.experimental.pallas import tpu_sc as plsc

C_DIM = 24
HID = 128
K = 20
NB = 5

Q = 256
NPAD = 10240
NY = 2048

SC_NC = 2
SC_NS = 16
SC_NW = SC_NC * SC_NS
SC_CH = 128


def _knn_kernel(pt_ref, pc_ref, idx_ref):
    f32 = jnp.float32
    b = pl.program_id(0)
    pt3 = pt_ref[0]
    pc3 = pc_ref[0]

    pn = jnp.sum(pt3 * pt3, axis=0, keepdims=True)
    pcn = jnp.sum(pc3 * pc3, axis=1, keepdims=True)
    mm = jax.lax.dot(pc3, pt3, preferred_element_type=f32)
    d2 = (pcn + pn) - 2.0 * mm

    iota = jax.lax.broadcasted_iota(jnp.int32, (NY, Q), 0).astype(f32)
    fny = f32(NY)

    rows = []
    for _ in range(K):
        m = jnp.min(d2, axis=0, keepdims=True)
        ii = jnp.where(d2 == m, iota, fny)
        j = jnp.min(ii, axis=0, keepdims=True)
        d2 = jnp.where(ii == j, jnp.inf, d2)
        rows.append(j)
    idx = jnp.concatenate(rows, axis=0).astype(jnp.int32)
    idx_ref[0, 0] = idx + b * NY


def _tab_conv1_kernel(tab_ref, w1t_ref, t1_ref):
    t1_ref[0] = jax.lax.dot(tab_ref[0], w1t_ref[...],
                            preferred_element_type=jnp.float32)


SC_NBUF = 5


def _sc_gather(tab_hbm, idx_hbm, out_hbm, idx_v, rows_v, gsem, wsem):
    wid = lax.axis_index("s") * SC_NC + lax.axis_index("c")
    n_rows = out_hbm.shape[0]
    b_per_w = n_rows // SC_NW
    base = wid * b_per_w

    pltpu.sync_copy(idx_hbm.at[pl.ds(base, b_per_w)], idx_v)

    grp = SC_NBUF * SC_CH

    @pl.loop(0, b_per_w // grp)
    def _group(g):
        off = g * grp
        hs = [
            pltpu.async_copy(
                tab_hbm.at[idx_v.at[pl.ds(off + b * SC_CH, SC_CH)]],
                rows_v.at[b], gsem)
            for b in range(SC_NBUF)
        ]
        for h in hs:
            h.wait()
        ws = [
            pltpu.async_copy(
                rows_v.at[b],
                out_hbm.at[pl.ds(base + off + b * SC_CH, SC_CH)], wsem)
            for b in range(SC_NBUF)
        ]
        for w in ws:
            w.wait()


def _decode_kernel(g_ref, p_ref,
                   wp1_ref, b1_ref,
                   w2_ref, b2_ref, w3_ref, b3_ref,
                   fcp_w_ref, fcp_b_ref,
                   fcc_w_ref, fcc_b_ref,
                   fc0_w_ref, fc0_b_ref,
                   fc1_w_ref, fc1_b_ref,
                   fcout_w_ref, fcout_b_ref,
                   out_ref):
    f32 = jnp.float32
    p3 = p_ref[0]
    w2 = w2_ref[...]
    b2 = b2_ref[...]
    w3 = w3_ref[...]
    b3 = b3_ref[...]

    pterm = jax.lax.dot(p3, wp1_ref[...], preferred_element_type=f32) \
        + b1_ref[...]

    def lrelu(x):
        return jnp.where(x >= 0, x, 0.2 * x)

    c = jnp.full((Q, C_DIM), -jnp.inf, dtype=f32)
    for r in range(K):
        h = lrelu(g_ref[r] + pterm)
        h = lrelu(jax.lax.dot(h, w2, preferred_element_type=f32) + b2)
        h = lrelu(jax.lax.dot(h, w3, preferred_element_type=f32) + b3)
        c = jnp.maximum(c, h)

    net = jax.lax.dot(p3, fcp_w_ref[...], preferred_element_type=f32) \
        + fcp_b_ref[...]
    for i in range(NB):
        net = net + jax.lax.dot(c, fcc_w_ref[i],
                                preferred_element_type=f32) + fcc_b_ref[i]
        hmid = jax.lax.dot(jax.nn.relu(net), fc0_w_ref[i],
                           preferred_element_type=f32) + fc0_b_ref[i]
        dx = jax.lax.dot(jax.nn.relu(hmid), fc1_w_ref[i],
                         preferred_element_type=f32) + fc1_b_ref[i]
        net = net + dx
    occ = jnp.sum(jax.nn.relu(net) * fcout_w_ref[...], axis=1,
                  keepdims=True) + fcout_b_ref[...]
    out_ref[0, 0] = occ


NCHUNK = 4


@jax.jit
def kernel(p, pc, feat, params):
    f32 = jnp.float32
    P = params
    bs, nx, _ = p.shape

    p_pad = jnp.zeros((bs, NPAD, 3), f32).at[:, :nx].set(p)
    p_t = jnp.transpose(p_pad, (0, 2, 1))

    def bn_scale_shift(name):
        s = P[name + "_gamma"] / jnp.sqrt(P[name + "_var"] + 1e-5)
        t = P[name + "_beta"] - P[name + "_mean"] * s
        return s, t

    s1, tb1 = bn_scale_shift("bn1")
    s2, t2 = bn_scale_shift("bn2")
    s3, t3 = bn_scale_shift("bn3")

    w1 = P["conv1_W"].T * s1[None, :]
    w1y, w1x, w1f = w1[0:3], w1[3:6], w1[6:30]
    w1t = jnp.zeros((32, HID), f32).at[0:3].set(w1y).at[3:27].set(w1f)
    wp1 = w1x - w1y
    b1 = tb1[None, :]
    w2 = P["conv2_W"].T * s2[None, :]
    b2 = t2[None, :]
    w3 = P["conv3_W"].T * s3[None, :]
    b3 = t3[None, :]

    tab = jnp.zeros((bs, NY, 32), f32)
    tab = tab.at[:, :, 0:3].set(pc).at[:, :, 3:27].set(feat)
    t1_all = pl.pallas_call(
        _tab_conv1_kernel,
        grid=(bs,),
        in_specs=[
            pl.BlockSpec((1, NY, 32), lambda b: (b, 0, 0)),
            pl.BlockSpec((32, HID), lambda b: (0, 0)),
        ],
        out_specs=pl.BlockSpec((1, NY, HID), lambda b: (b, 0, 0)),
        out_shape=jax.ShapeDtypeStruct((bs, NY, HID), f32),
    )(tab, w1t)
    tab_flat = t1_all.reshape(bs * NY, HID)

    mesh = plsc.VectorSubcoreMesh(core_axis_name="c", subcore_axis_name="s",
                                  num_cores=SC_NC, num_subcores=SC_NS)

    npc = NPAD // NCHUNK
    nblk = npc // Q
    n_edges = bs * npc * K

    def whole(shape):
        n = len(shape)
        return pl.BlockSpec(shape, lambda b, i: (0,) * n)

    outs = []
    for ci in range(NCHUNK):
        p_t_c = lax.slice_in_dim(p_t, ci * npc, (ci + 1) * npc, axis=2)
        idx = pl.pallas_call(
            _knn_kernel,
            grid=(bs, nblk),
            in_specs=[
                pl.BlockSpec((1, 3, Q), lambda b, i: (b, 0, i)),
                pl.BlockSpec((1, NY, 3), lambda b, i: (b, 0, 0)),
            ],
            out_specs=pl.BlockSpec((1, 1, K, Q), lambda b, i: (b, i, 0, 0)),
            out_shape=jax.ShapeDtypeStruct((bs, nblk, K, Q), jnp.int32),
        )(p_t_c, pc)

        gathered = pl.kernel(
            _sc_gather,
            out_type=jax.ShapeDtypeStruct((n_edges, HID), f32),
            mesh=mesh,
            scratch_types=[
                pltpu.VMEM((n_edges // SC_NW,), jnp.int32),
                pltpu.VMEM((SC_NBUF, SC_CH, HID), f32),
                pltpu.SemaphoreType.DMA,
                pltpu.SemaphoreType.DMA,
            ],
        )(tab_flat, idx.reshape(n_edges))
        g4 = gathered.reshape(bs * nblk * K, Q, HID)
        p_pad_c = lax.slice_in_dim(p_pad, ci * npc, (ci + 1) * npc, axis=1)

        nb = nblk
        out_c = pl.pallas_call(
            _decode_kernel,
            grid=(bs, nblk),
            in_specs=[
                pl.BlockSpec((K, Q, HID),
                             lambda b, i, nb=nb: (b * nb + i, 0, 0)),
                pl.BlockSpec((1, Q, 3), lambda b, i: (b, i, 0)),
                whole((3, HID)), whole((1, HID)),
                whole((HID, HID)), whole((1, HID)),
                whole((HID, C_DIM)), whole((1, C_DIM)),
                whole((3, HID)), whole((1, HID)),
                whole((NB, C_DIM, HID)), whole((NB, 1, HID)),
                whole((NB, HID, HID)), whole((NB, 1, HID)),
                whole((NB, HID, HID)), whole((NB, 1, HID)),
                whole((1, HID)), whole((1, 1)),
            ],
            out_specs=pl.BlockSpec((1, 1, Q, 1), lambda b, i: (b, i, 0, 0)),
            out_shape=jax.ShapeDtypeStruct((bs, nblk, Q, 1), f32),
        )(
            g4, p_pad_c,
            wp1, b1, w2, b2, w3, b3,
            P["fcp_W"], P["fcp_b"][None, :],
            P["fcc_W"], P["fcc_b"][:, None, :],
            P["fc0_W"], P["fc0_b"][:, None, :],
            P["fc1_W"], P["fc1_b"][:, None, :],
            P["fcout_W"].T, P["fcout_b"][None, :],
        )
        outs.append(out_c.reshape(bs, npc))

    return jnp.concatenate(outs, axis=1)[:, :nx]


# speedup vs baseline: 1.1155x; 1.1155x over previous
import functools

import jax
import jax.numpy as jnp
from jax import lax
from jax.experimental import pallas as pl
from jax.experimental.pallas import tpu as pltpu
from jax.experimental.pallas import tpu_sc as plsc

C_DIM = 24
HID = 128
K = 20
NB = 5

Q = 256
NPAD = 10240
NY = 2048

SC_NC = 2
SC_NS = 16
SC_NW = SC_NC * SC_NS
SC_CH = 128


def _knn_kernel(pt_ref, pc_ref, idx_ref):
    f32 = jnp.float32
    b = pl.program_id(0)
    pt3 = pt_ref[0]
    pc3 = pc_ref[0]

    pn = jnp.sum(pt3 * pt3, axis=0, keepdims=True)
    pcn = jnp.sum(pc3 * pc3, axis=1, keepdims=True)
    mm = jax.lax.dot(pc3, pt3, preferred_element_type=f32)
    d2 = (pcn + pn) - 2.0 * mm

    iota = jax.lax.broadcasted_iota(jnp.int32, (NY, Q), 0).astype(f32)
    fny = f32(NY)

    rows = []
    for _ in range(K):
        m = jnp.min(d2, axis=0, keepdims=True)
        ii = jnp.where(d2 == m, iota, fny)
        j = jnp.min(ii, axis=0, keepdims=True)
        d2 = jnp.where(d2 == m, jnp.inf, d2)
        rows.append(j)
    idx = jnp.concatenate(rows, axis=0).astype(jnp.int32)
    idx_ref[0, 0] = idx + b * NY


def _tab_conv1_kernel(tab_ref, w1t_ref, t1_ref):
    t1_ref[0] = jax.lax.dot(tab_ref[0], w1t_ref[...],
                            preferred_element_type=jnp.float32)


SC_NBUF = 5


def _sc_gather(tab_hbm, idx_hbm, out_hbm, idx_v, rows_v, gsem, wsem):
    wid = lax.axis_index("s") * SC_NC + lax.axis_index("c")
    n_rows = out_hbm.shape[0]
    b_per_w = n_rows // SC_NW
    base = wid * b_per_w

    pltpu.sync_copy(idx_hbm.at[pl.ds(base, b_per_w)], idx_v)

    grp = SC_NBUF * SC_CH

    @pl.loop(0, b_per_w // grp)
    def _group(g):
        off = g * grp
        hs = [
            pltpu.async_copy(
                tab_hbm.at[idx_v.at[pl.ds(off + b * SC_CH, SC_CH)]],
                rows_v.at[b], gsem)
            for b in range(SC_NBUF)
        ]
        for h in hs:
            h.wait()
        ws = [
            pltpu.async_copy(
                rows_v.at[b],
                out_hbm.at[pl.ds(base + off + b * SC_CH, SC_CH)], wsem)
            for b in range(SC_NBUF)
        ]
        for w in ws:
            w.wait()


def _decode_kernel(g_ref, p_ref,
                   wp1_ref, b1_ref,
                   w2_ref, b2_ref, w3_ref, b3_ref,
                   fcp_w_ref, fcp_b_ref,
                   fcc_w_ref, fcc_b_ref,
                   fc0_w_ref, fc0_b_ref,
                   fc1_w_ref, fc1_b_ref,
                   fcout_w_ref, fcout_b_ref,
                   out_ref):
    f32 = jnp.float32
    p3 = p_ref[0]
    w2 = w2_ref[...]
    b2 = b2_ref[...]
    w3 = w3_ref[...]
    b3 = b3_ref[...]

    pterm = jax.lax.dot(p3, wp1_ref[...], preferred_element_type=f32) \
        + b1_ref[...]

    def lrelu(x):
        return jnp.where(x >= 0, x, 0.2 * x)

    c = jnp.full((Q, C_DIM), -jnp.inf, dtype=f32)
    for r in range(K):
        h = lrelu(g_ref[r] + pterm)
        h = lrelu(jax.lax.dot(h, w2, preferred_element_type=f32) + b2)
        h = lrelu(jax.lax.dot(h, w3, preferred_element_type=f32) + b3)
        c = jnp.maximum(c, h)

    net = jax.lax.dot(p3, fcp_w_ref[...], preferred_element_type=f32) \
        + fcp_b_ref[...]
    for i in range(NB):
        net = net + jax.lax.dot(c, fcc_w_ref[i],
                                preferred_element_type=f32) + fcc_b_ref[i]
        hmid = jax.lax.dot(jax.nn.relu(net), fc0_w_ref[i],
                           preferred_element_type=f32) + fc0_b_ref[i]
        dx = jax.lax.dot(jax.nn.relu(hmid), fc1_w_ref[i],
                         preferred_element_type=f32) + fc1_b_ref[i]
        net = net + dx
    occ = jnp.sum(jax.nn.relu(net) * fcout_w_ref[...], axis=1,
                  keepdims=True) + fcout_b_ref[...]
    out_ref[0, 0] = occ


CHUNK_NBLK = (8, 24, 8)


@jax.jit
def kernel(p, pc, feat, params):
    f32 = jnp.float32
    P = params
    bs, nx, _ = p.shape

    p_pad = jnp.zeros((bs, NPAD, 3), f32).at[:, :nx].set(p)
    p_t = jnp.transpose(p_pad, (0, 2, 1))

    def bn_scale_shift(name):
        s = P[name + "_gamma"] / jnp.sqrt(P[name + "_var"] + 1e-5)
        t = P[name + "_beta"] - P[name + "_mean"] * s
        return s, t

    s1, tb1 = bn_scale_shift("bn1")
    s2, t2 = bn_scale_shift("bn2")
    s3, t3 = bn_scale_shift("bn3")

    w1 = P["conv1_W"].T * s1[None, :]
    w1y, w1x, w1f = w1[0:3], w1[3:6], w1[6:30]
    w1t = jnp.zeros((32, HID), f32).at[0:3].set(w1y).at[3:27].set(w1f)
    wp1 = w1x - w1y
    b1 = tb1[None, :]
    w2 = P["conv2_W"].T * s2[None, :]
    b2 = t2[None, :]
    w3 = P["conv3_W"].T * s3[None, :]
    b3 = t3[None, :]

    tab = jnp.zeros((bs, NY, 32), f32)
    tab = tab.at[:, :, 0:3].set(pc).at[:, :, 3:27].set(feat)
    t1_all = pl.pallas_call(
        _tab_conv1_kernel,
        grid=(bs,),
        in_specs=[
            pl.BlockSpec((1, NY, 32), lambda b: (b, 0, 0)),
            pl.BlockSpec((32, HID), lambda b: (0, 0)),
        ],
        out_specs=pl.BlockSpec((1, NY, HID), lambda b: (b, 0, 0)),
        out_shape=jax.ShapeDtypeStruct((bs, NY, HID), f32),
    )(tab, w1t)
    tab_flat = t1_all.reshape(bs * NY, HID)

    mesh = plsc.VectorSubcoreMesh(core_axis_name="c", subcore_axis_name="s",
                                  num_cores=SC_NC, num_subcores=SC_NS)

    def whole(shape):
        n = len(shape)
        return pl.BlockSpec(shape, lambda b, i: (0,) * n)

    outs = []
    qoff = 0
    for nblk in CHUNK_NBLK:
        npc = nblk * Q
        n_edges = bs * npc * K
        p_t_c = lax.slice_in_dim(p_t, qoff, qoff + npc, axis=2)
        idx = pl.pallas_call(
            _knn_kernel,
            grid=(bs, nblk),
            in_specs=[
                pl.BlockSpec((1, 3, Q), lambda b, i: (b, 0, i)),
                pl.BlockSpec((1, NY, 3), lambda b, i: (b, 0, 0)),
            ],
            out_specs=pl.BlockSpec((1, 1, K, Q), lambda b, i: (b, i, 0, 0)),
            out_shape=jax.ShapeDtypeStruct((bs, nblk, K, Q), jnp.int32),
        )(p_t_c, pc)

        gathered = pl.kernel(
            _sc_gather,
            out_type=jax.ShapeDtypeStruct((n_edges, HID), f32),
            mesh=mesh,
            scratch_types=[
                pltpu.VMEM((n_edges // SC_NW,), jnp.int32),
                pltpu.VMEM((SC_NBUF, SC_CH, HID), f32),
                pltpu.SemaphoreType.DMA,
                pltpu.SemaphoreType.DMA,
            ],
        )(tab_flat, idx.reshape(n_edges))
        g4 = gathered.reshape(bs * nblk * K, Q, HID)
        p_pad_c = lax.slice_in_dim(p_pad, qoff, qoff + npc, axis=1)

        nb = nblk
        out_c = pl.pallas_call(
            _decode_kernel,
            grid=(bs, nblk),
            in_specs=[
                pl.BlockSpec((K, Q, HID),
                             lambda b, i, nb=nb: (b * nb + i, 0, 0)),
                pl.BlockSpec((1, Q, 3), lambda b, i: (b, i, 0)),
                whole((3, HID)), whole((1, HID)),
                whole((HID, HID)), whole((1, HID)),
                whole((HID, C_DIM)), whole((1, C_DIM)),
                whole((3, HID)), whole((1, HID)),
                whole((NB, C_DIM, HID)), whole((NB, 1, HID)),
                whole((NB, HID, HID)), whole((NB, 1, HID)),
                whole((NB, HID, HID)), whole((NB, 1, HID)),
                whole((1, HID)), whole((1, 1)),
            ],
            out_specs=pl.BlockSpec((1, 1, Q, 1), lambda b, i: (b, i, 0, 0)),
            out_shape=jax.ShapeDtypeStruct((bs, nblk, Q, 1), f32),
        )(
            g4, p_pad_c,
            wp1, b1, w2, b2, w3, b3,
            P["fcp_W"], P["fcp_b"][None, :],
            P["fcc_W"], P["fcc_b"][:, None, :],
            P["fc0_W"], P["fc0_b"][:, None, :],
            P["fc1_W"], P["fc1_b"][:, None, :],
            P["fcout_W"].T, P["fcout_b"][None, :],
        )
        outs.append(out_c.reshape(bs, npc))
        qoff += npc

    return jnp.concatenate(outs, axis=1)[:, :nx]


# speedup vs baseline: 1.1241x; 1.0076x over previous
import functools

import jax
import jax.numpy as jnp
from jax import lax
from jax.experimental import pallas as pl
from jax.experimental.pallas import tpu as pltpu
from jax.experimental.pallas import tpu_sc as plsc

C_DIM = 24
HID = 128
K = 20
NB = 5

Q = 256
NPAD = 10240
NY = 2048

SC_NC = 2
SC_NS = 16
SC_NW = SC_NC * SC_NS
SC_CH = 128


def _knn_kernel(pt_ref, pc_ref, idx_ref):
    f32 = jnp.float32
    b = pl.program_id(0)
    pt3 = pt_ref[0]
    pc3 = pc_ref[0]

    pn = jnp.sum(pt3 * pt3, axis=0, keepdims=True)
    pcn = jnp.sum(pc3 * pc3, axis=1, keepdims=True)
    mm = jax.lax.dot(pc3, pt3, preferred_element_type=f32)
    d2 = (pcn + pn) - 2.0 * mm

    iota = jax.lax.broadcasted_iota(jnp.int32, (NY, Q), 0).astype(f32)
    fny = f32(NY)

    rows = []
    for _ in range(K):
        m = jnp.min(d2, axis=0, keepdims=True)
        ii = jnp.where(d2 == m, iota, fny)
        j = jnp.min(ii, axis=0, keepdims=True)
        d2 = jnp.where(d2 == m, jnp.inf, d2)
        rows.append(j)
    idx = jnp.concatenate(rows, axis=0).astype(jnp.int32)
    idx_ref[0, 0] = idx + b * NY


def _tab_conv1_kernel(tab_ref, w1t_ref, t1_ref):
    t1_ref[0] = jax.lax.dot(tab_ref[0], w1t_ref[...],
                            preferred_element_type=jnp.float32)


SC_NBUF = 5


def _sc_gather(tab_hbm, idx_hbm, out_hbm, idx_v, rows_v, gsem, wsem):
    wid = lax.axis_index("s") * SC_NC + lax.axis_index("c")
    n_rows = out_hbm.shape[0]
    b_per_w = n_rows // SC_NW
    base = wid * b_per_w

    pltpu.sync_copy(idx_hbm.at[pl.ds(base, b_per_w)], idx_v)

    grp = SC_NBUF * SC_CH

    @pl.loop(0, b_per_w // grp)
    def _group(g):
        off = g * grp
        hs = [
            pltpu.async_copy(
                tab_hbm.at[idx_v.at[pl.ds(off + b * SC_CH, SC_CH)]],
                rows_v.at[b], gsem)
            for b in range(SC_NBUF)
        ]
        for h in hs:
            h.wait()
        ws = [
            pltpu.async_copy(
                rows_v.at[b],
                out_hbm.at[pl.ds(base + off + b * SC_CH, SC_CH)], wsem)
            for b in range(SC_NBUF)
        ]
        for w in ws:
            w.wait()


def _decode_kernel(g_ref, p_ref,
                   wp1_ref, b1_ref,
                   w2_ref, b2_ref, w3_ref, b3_ref,
                   fcp_w_ref, fcp_b_ref,
                   fcc_w_ref, fcc_b_ref,
                   fc0_w_ref, fc0_b_ref,
                   fc1_w_ref, fc1_b_ref,
                   fcout_w_ref, fcout_b_ref,
                   out_ref):
    f32 = jnp.float32
    p3 = p_ref[0]
    w2 = w2_ref[...]
    b2 = b2_ref[...]
    w3 = w3_ref[...]
    b3 = b3_ref[...]

    pterm = jax.lax.dot(p3, wp1_ref[...], preferred_element_type=f32) \
        + b1_ref[...]

    def lrelu(x):
        return jnp.where(x >= 0, x, 0.2 * x)

    c = jnp.full((Q, C_DIM), -jnp.inf, dtype=f32)
    for r in range(K):
        h = lrelu(g_ref[r] + pterm)
        h = lrelu(jax.lax.dot(h, w2, preferred_element_type=f32) + b2)
        h = lrelu(jax.lax.dot(h, w3, preferred_element_type=f32) + b3)
        c = jnp.maximum(c, h)

    net = jax.lax.dot(p3, fcp_w_ref[...], preferred_element_type=f32) \
        + fcp_b_ref[...]
    for i in range(NB):
        net = net + jax.lax.dot(c, fcc_w_ref[i],
                                preferred_element_type=f32) + fcc_b_ref[i]
        hmid = jax.lax.dot(jax.nn.relu(net), fc0_w_ref[i],
                           preferred_element_type=f32) + fc0_b_ref[i]
        dx = jax.lax.dot(jax.nn.relu(hmid), fc1_w_ref[i],
                         preferred_element_type=f32) + fc1_b_ref[i]
        net = net + dx
    occ = jnp.sum(jax.nn.relu(net) * fcout_w_ref[...], axis=1,
                  keepdims=True) + fcout_b_ref[...]
    out_ref[0, 0] = occ


CHUNK_NBLK = (4, 16, 16, 4)


@jax.jit
def kernel(p, pc, feat, params):
    f32 = jnp.float32
    P = params
    bs, nx, _ = p.shape

    p_pad = jnp.zeros((bs, NPAD, 3), f32).at[:, :nx].set(p)
    p_t = jnp.transpose(p_pad, (0, 2, 1))

    def bn_scale_shift(name):
        s = P[name + "_gamma"] / jnp.sqrt(P[name + "_var"] + 1e-5)
        t = P[name + "_beta"] - P[name + "_mean"] * s
        return s, t

    s1, tb1 = bn_scale_shift("bn1")
    s2, t2 = bn_scale_shift("bn2")
    s3, t3 = bn_scale_shift("bn3")

    w1 = P["conv1_W"].T * s1[None, :]
    w1y, w1x, w1f = w1[0:3], w1[3:6], w1[6:30]
    w1t = jnp.zeros((32, HID), f32).at[0:3].set(w1y).at[3:27].set(w1f)
    wp1 = w1x - w1y
    b1 = tb1[None, :]
    w2 = P["conv2_W"].T * s2[None, :]
    b2 = t2[None, :]
    w3 = P["conv3_W"].T * s3[None, :]
    b3 = t3[None, :]

    tab = jnp.zeros((bs, NY, 32), f32)
    tab = tab.at[:, :, 0:3].set(pc).at[:, :, 3:27].set(feat)
    t1_all = pl.pallas_call(
        _tab_conv1_kernel,
        grid=(bs,),
        in_specs=[
            pl.BlockSpec((1, NY, 32), lambda b: (b, 0, 0)),
            pl.BlockSpec((32, HID), lambda b: (0, 0)),
        ],
        out_specs=pl.BlockSpec((1, NY, HID), lambda b: (b, 0, 0)),
        out_shape=jax.ShapeDtypeStruct((bs, NY, HID), f32),
    )(tab, w1t)
    tab_flat = t1_all.reshape(bs * NY, HID)

    mesh = plsc.VectorSubcoreMesh(core_axis_name="c", subcore_axis_name="s",
                                  num_cores=SC_NC, num_subcores=SC_NS)

    def whole(shape):
        n = len(shape)
        return pl.BlockSpec(shape, lambda b, i: (0,) * n)

    outs = []
    qoff = 0
    for nblk in CHUNK_NBLK:
        npc = nblk * Q
        n_edges = bs * npc * K
        p_t_c = lax.slice_in_dim(p_t, qoff, qoff + npc, axis=2)
        idx = pl.pallas_call(
            _knn_kernel,
            grid=(bs, nblk),
            in_specs=[
                pl.BlockSpec((1, 3, Q), lambda b, i: (b, 0, i)),
                pl.BlockSpec((1, NY, 3), lambda b, i: (b, 0, 0)),
            ],
            out_specs=pl.BlockSpec((1, 1, K, Q), lambda b, i: (b, i, 0, 0)),
            out_shape=jax.ShapeDtypeStruct((bs, nblk, K, Q), jnp.int32),
        )(p_t_c, pc)

        gathered = pl.kernel(
            _sc_gather,
            out_type=jax.ShapeDtypeStruct((n_edges, HID), f32),
            mesh=mesh,
            scratch_types=[
                pltpu.VMEM((n_edges // SC_NW,), jnp.int32),
                pltpu.VMEM((SC_NBUF, SC_CH, HID), f32),
                pltpu.SemaphoreType.DMA,
                pltpu.SemaphoreType.DMA,
            ],
        )(tab_flat, idx.reshape(n_edges))
        g4 = gathered.reshape(bs * nblk * K, Q, HID)
        p_pad_c = lax.slice_in_dim(p_pad, qoff, qoff + npc, axis=1)

        nb = nblk
        out_c = pl.pallas_call(
            _decode_kernel,
            grid=(bs, nblk),
            in_specs=[
                pl.BlockSpec((K, Q, HID),
                             lambda b, i, nb=nb: (b * nb + i, 0, 0)),
                pl.BlockSpec((1, Q, 3), lambda b, i: (b, i, 0)),
                whole((3, HID)), whole((1, HID)),
                whole((HID, HID)), whole((1, HID)),
                whole((HID, C_DIM)), whole((1, C_DIM)),
                whole((3, HID)), whole((1, HID)),
                whole((NB, C_DIM, HID)), whole((NB, 1, HID)),
                whole((NB, HID, HID)), whole((NB, 1, HID)),
                whole((NB, HID, HID)), whole((NB, 1, HID)),
                whole((1, HID)), whole((1, 1)),
            ],
            out_specs=pl.BlockSpec((1, 1, Q, 1), lambda b, i: (b, i, 0, 0)),
            out_shape=jax.ShapeDtypeStruct((bs, nblk, Q, 1), f32),
        )(
            g4, p_pad_c,
            wp1, b1, w2, b2, w3, b3,
            P["fcp_W"], P["fcp_b"][None, :],
            P["fcc_W"], P["fcc_b"][:, None, :],
            P["fc0_W"], P["fc0_b"][:, None, :],
            P["fc1_W"], P["fc1_b"][:, None, :],
            P["fcout_W"].T, P["fcout_b"][None, :],
        )
        outs.append(out_c.reshape(bs, npc))
        qoff += npc

    return jnp.concatenate(outs, axis=1)[:, :nx]
